# baseline (speedup 1.0000x reference)
import jax
import jax.numpy as jnp
from jax.experimental import pallas as pl

_S0, _S1, _C = 64, 64, 8
_F = 32
_P = _S0 * _S1
_GID = [((i * 7) % 64) * _S1 + (i * 13) % 64 for i in range(_F)]


def _body(x4_ref, w_ref, r_ref, outT_ref):
    x4 = x4_ref[...]
    RT = r_ref[...].reshape(256, 16, 8).transpose(2, 0, 1).reshape(8, _P)
    g_rows = [x4[g // 4:g // 4 + 1, (g % 4) * 32:(g % 4) * 32 + 32] for g in _GID]
    gathered = jnp.concatenate(g_rows, axis=0)
    r_cols = [RT[:, g:g + 1] for g in _GID]
    r_stT = jnp.concatenate(r_cols, axis=1)
    proj = jnp.dot(gathered, w_ref[...], preferred_element_type=jnp.float32)
    U = jnp.exp(RT)
    Ui = jnp.exp(-RT)
    vT = jnp.exp(r_stT)
    viT = jnp.exp(-r_stT)
    term = jnp.minimum(viT[:, :, None] * U[:, None, :],
                       vT[:, :, None] * Ui[:, None, :])
    dT = jnp.sum(term, axis=0)
    outT_ref[...] = jax.lax.dot_general(proj, dT, (((0,), (0,)), ((), ())),
                                        preferred_element_type=jnp.float32)


def kernel(x, W, R):
    x4 = x.reshape(_P // 4, _F * 4)
    R256 = R.reshape(256, 128)
    outT = pl.pallas_call(
        _body,
        out_shape=jax.ShapeDtypeStruct((_F, _P), jnp.float32),
    )(x4, W, R256)
    return outT.T.reshape(_S0, _S1, _F)


# speedup vs baseline: 1.2727x; 1.2727x over previous
import jax
import jax.numpy as jnp
from jax.experimental import pallas as pl

_S0, _S1, _C = 64, 64, 8
_F = 32
_P = _S0 * _S1
_GID = [((i * 7) % 64) * _S1 + (i * 13) % 64 for i in range(_F)]


def _body(x4_ref, w_ref, rt_ref, outT_ref):
    x4 = x4_ref[...]
    RT = rt_ref[...]
    g_rows = [x4[g // 4:g // 4 + 1, (g % 4) * 32:(g % 4) * 32 + 32] for g in _GID]
    gathered = jnp.concatenate(g_rows, axis=0)
    r_cols = [RT[:, g:g + 1] for g in _GID]
    r_stT = jnp.concatenate(r_cols, axis=1)
    proj = jnp.dot(gathered, w_ref[...], preferred_element_type=jnp.float32)
    U = jnp.exp(RT)
    Ui = jnp.exp(-RT)
    vT = jnp.exp(r_stT)
    viT = jnp.exp(-r_stT)
    term = jnp.minimum(viT[:, :, None] * U[:, None, :],
                       vT[:, :, None] * Ui[:, None, :])
    dT = jnp.sum(term, axis=0)
    outT_ref[...] = jax.lax.dot_general(proj, dT, (((0,), (0,)), ((), ())),
                                        preferred_element_type=jnp.float32)


def kernel(x, W, R):
    x4 = x.reshape(_P // 4, _F * 4)
    RT = R.reshape(_P, _C).T
    outT = pl.pallas_call(
        _body,
        out_shape=jax.ShapeDtypeStruct((_F, _P), jnp.float32),
    )(x4, W, RT)
    return outT.T.reshape(_S0, _S1, _F)
